# baseline (speedup 1.0000x reference)
import functools

import jax
import jax.numpy as jnp
from jax import lax
from jax.experimental import pallas as pl
from jax.experimental.pallas import tpu as pltpu
from jax.experimental.pallas import tpu_sc as plsc

NC = 2
NS = 16
NW = NC * NS
MAXLEN = 200
EMBED = 64
BATCH = 4096
CHUNK = MAXLEN // 2
NCHUNKS = BATCH * 2
CPW = NCHUNKS // NW
NBUF = 4


def _emb_body(x2_hbm, tab_hbm, pos_hbm, out_hbm, idx_v, pos_v,
              b0, b1, b2, b3, g0, g1, g2, g3, o0, o1, o2, o3):
    bufs = (b0, b1, b2, b3)
    gsems = (g0, g1, g2, g3)
    osems = (o0, o1, o2, o3)

    wid = lax.axis_index("s") * NC + lax.axis_index("c")
    row0 = wid * CPW

    pltpu.sync_copy(pos_hbm, pos_v)
    pltpu.sync_copy(x2_hbm.at[pl.ds(row0, CPW)], idx_v)

    def gstart(c, p):
        pltpu.make_async_copy(tab_hbm.at[idx_v.at[c]], bufs[p], gsems[p]).start()

    def gwait(c, p):
        pltpu.make_async_copy(tab_hbm.at[idx_v.at[c]], bufs[p], gsems[p]).wait()

    def odesc(c, p, off):
        seq = (row0 + c) // 2
        dst = out_hbm.at[seq, pl.ds(off, CHUNK)]
        return pltpu.make_async_copy(bufs[p], dst, osems[p])

    def off_of(k):
        return (k % 2) * CHUNK

    for p in range(NBUF - 1):
        gstart(p, p)

    @pl.loop(0, CPW, step=NBUF)
    def _(c):
        for k in range(NBUF):
            ck = c + k
            p = k
            pn = (k + NBUF - 1) % NBUF

            @pl.when(ck + NBUF - 1 < CPW)
            def _():
                @pl.when(ck >= 1)
                def _():
                    odesc(ck - 1, pn, off_of(k - 1)).wait()
                gstart(ck + NBUF - 1, pn)

            gwait(ck, p)
            buf = bufs[p]
            off = off_of(k)

            @plsc.parallel_loop(0, CHUNK, unroll=4)
            def _(r):
                for g in range(EMBED // 16):
                    s = pl.ds(g * 16, 16)
                    plsc.addupdate(buf.at[r, s], pos_v[off + r, s])

            odesc(ck, p, off).start()

    for k in range(NBUF):
        odesc(CPW - NBUF + k, k, off_of(k)).wait()


@jax.jit
def kernel(x, token_table, pos_table):
    x2 = x.reshape(NCHUNKS, CHUNK).astype(jnp.int32)

    mesh = plsc.VectorSubcoreMesh(core_axis_name="c", subcore_axis_name="s")
    run = pl.kernel(
        _emb_body,
        out_type=jax.ShapeDtypeStruct((BATCH, MAXLEN, EMBED), jnp.float32),
        mesh=mesh,
        scratch_types=(
            [pltpu.VMEM((CPW, CHUNK), jnp.int32),
             pltpu.VMEM((MAXLEN, EMBED), jnp.float32)]
            + [pltpu.VMEM((CHUNK, EMBED), jnp.float32)] * NBUF
            + [pltpu.SemaphoreType.DMA] * (2 * NBUF)
        ),
        compiler_params=pltpu.CompilerParams(use_tc_tiling_on_sc=False),
    )
    return run(x2, token_table, pos_table)


# speedup vs baseline: 1.6787x; 1.6787x over previous
import functools

import jax
import jax.numpy as jnp
from jax import lax
from jax.experimental import pallas as pl
from jax.experimental.pallas import tpu as pltpu
from jax.experimental.pallas import tpu_sc as plsc

NC = 2
NS = 16
NW = NC * NS
MAXLEN = 200
EMBED = 64
BATCH = 4096
LANES = 128
CHUNK = MAXLEN // 2
BFLAT = BATCH * MAXLEN
NCHUNKS = BATCH * 2
CPW = NCHUNKS // NW
NBUF = 4


def _emb_body(x2_hbm, tab_hbm, pos_hbm, out_hbm, idx_v, pos_v,
              b0, b1, b2, b3, g0, g1, g2, g3, o0, o1, o2, o3):
    bufs = (b0, b1, b2, b3)
    gsems = (g0, g1, g2, g3)
    osems = (o0, o1, o2, o3)

    wid = lax.axis_index("s") * NC + lax.axis_index("c")
    row0 = wid * CPW

    pltpu.sync_copy(pos_hbm, pos_v)
    pltpu.sync_copy(x2_hbm.at[pl.ds(row0, CPW)], idx_v)

    def gstart(c, p):
        pltpu.make_async_copy(tab_hbm.at[idx_v.at[c]], bufs[p], gsems[p]).start()

    def gwait(c, p):
        pltpu.make_async_copy(tab_hbm.at[idx_v.at[c]], bufs[p], gsems[p]).wait()

    def odesc(c, p):
        dst = out_hbm.at[pl.ds((row0 + c) * CHUNK, CHUNK), pl.ds(0, EMBED)]
        return pltpu.make_async_copy(bufs[p], dst, osems[p])

    def off_of(k):
        return (k % 2) * CHUNK

    for p in range(NBUF - 1):
        gstart(p, p)

    @pl.loop(0, CPW, step=NBUF)
    def _(c):
        for k in range(NBUF):
            ck = c + k
            p = k
            pn = (k + NBUF - 1) % NBUF

            @pl.when(ck + NBUF - 1 < CPW)
            def _():
                @pl.when(ck >= 1)
                def _():
                    odesc(ck - 1, pn).wait()
                gstart(ck + NBUF - 1, pn)

            gwait(ck, p)
            buf = bufs[p]
            off = off_of(k)

            @plsc.parallel_loop(0, CHUNK, unroll=4)
            def _(r):
                for g in range(EMBED // 16):
                    s = pl.ds(g * 16, 16)
                    plsc.addupdate(buf.at[r, s], pos_v[off + r, s])

            odesc(ck, p).start()

    for k in range(NBUF):
        odesc(CPW - NBUF + k, k).wait()


@jax.jit
def kernel(x, token_table, pos_table):
    x2 = x.reshape(NCHUNKS, CHUNK).astype(jnp.int32)

    mesh = plsc.VectorSubcoreMesh(core_axis_name="c", subcore_axis_name="s")
    run = pl.kernel(
        _emb_body,
        out_type=jax.ShapeDtypeStruct((BFLAT, LANES), jnp.float32),
        mesh=mesh,
        scratch_types=(
            [pltpu.VMEM((CPW, CHUNK), jnp.int32),
             pltpu.VMEM((MAXLEN, EMBED), jnp.float32)]
            + [pltpu.VMEM((CHUNK, EMBED), jnp.float32)] * NBUF
            + [pltpu.SemaphoreType.DMA] * (2 * NBUF)
        ),
        compiler_params=pltpu.CompilerParams(use_tc_tiling_on_sc=False),
    )
    out128 = run(x2, token_table, pos_table)
    return out128[:, :EMBED].reshape(BATCH, MAXLEN, EMBED)
